# Initial kernel scaffold; baseline (speedup 1.0000x reference)
import functools

import jax
import jax.numpy as jnp
from jax import lax
from jax.experimental import pallas as pl
from jax.experimental.pallas import tpu as pltpu
from jax.experimental.pallas import tpu_sc as plsc

N_NODES = 10000
N_EDGES = 320000
D = 128
D_AUG = 144

NC = 2
NS = 16
NW = NC * NS
EDGES_PER_TILE = N_EDGES // NW
CHUNK = 80
CHUNKS_PER_TILE = EDGES_PER_TILE // CHUNK
ROWS_PER_TILE = N_NODES // NS
ZROWS = 125
BM = 400


def _make_sc_segsum(d_cols):
    mesh = plsc.VectorSubcoreMesh(core_axis_name="c", subcore_axis_name="s")

    @functools.partial(
        pl.kernel,
        mesh=mesh,
        out_type=jax.ShapeDtypeStruct((NC, N_NODES, d_cols), jnp.float32),
        scratch_types=[
            pltpu.VMEM((CHUNKS_PER_TILE, CHUNK), jnp.int32),
            pltpu.VMEM((CHUNKS_PER_TILE, CHUNK), jnp.int32),
            pltpu.VMEM((CHUNK, d_cols), jnp.float32),
            pltpu.VMEM((ZROWS, d_cols), jnp.float32),
            pltpu.VMEM_SHARED((N_NODES, d_cols), jnp.float32),
            pltpu.SemaphoreType.DMA,
        ],
    )
    def sc_segsum(table_hbm, src_hbm, dst_hbm, out_hbm,
                  src_v, dst_v, rows_v, zbuf, acc, sem):
        c = lax.axis_index("c")
        s = lax.axis_index("s")
        wid = s * NC + c

        pltpu.sync_copy(src_hbm.at[pl.ds(wid * CHUNKS_PER_TILE, CHUNKS_PER_TILE)],
                        src_v)
        pltpu.sync_copy(dst_hbm.at[pl.ds(wid * CHUNKS_PER_TILE, CHUNKS_PER_TILE)],
                        dst_v)

        def _zrow(i, carry):
            for k in range(d_cols // 16):
                zbuf[i, pl.ds(k * 16, 16)] = jnp.zeros((16,), jnp.float32)
            return carry
        lax.fori_loop(0, ZROWS, _zrow, 0)

        def _zcopy(b, carry):
            off = (s * (ROWS_PER_TILE // ZROWS) + b) * ZROWS
            pltpu.sync_copy(zbuf, acc.at[pl.ds(off, ZROWS)])
            return carry
        lax.fori_loop(0, ROWS_PER_TILE // ZROWS, _zcopy, 0)
        plsc.subcore_barrier()

        def _chunk(j, carry):
            pltpu.async_copy(table_hbm.at[src_v.at[j]], rows_v, sem).wait()
            pltpu.sync_copy(rows_v, acc.at[dst_v.at[j]], add=True)
            return carry
        lax.fori_loop(0, CHUNKS_PER_TILE, _chunk, 0)
        plsc.subcore_barrier()

        pltpu.sync_copy(acc.at[pl.ds(s * ROWS_PER_TILE, ROWS_PER_TILE)],
                        out_hbm.at[c, pl.ds(s * ROWS_PER_TILE, ROWS_PER_TILE)])

    return sc_segsum


_sc_segsum_aug = _make_sc_segsum(D_AUG)
_sc_segsum_plain = _make_sc_segsum(D)


def _tc1_body(x_ref, s_ref, At_ref, Bt_ref, b_ref, o_ref):
    sblk = s_ref[0] + s_ref[1]
    cnt = sblk[:, D:D + 1]
    feat = sblk[:, :D]
    inv = jnp.where(cnt > 0.0, 1.0 / jnp.maximum(cnt, 1.0), 0.0)
    mm = jnp.dot(x_ref[...], At_ref[...], preferred_element_type=jnp.float32)
    mm2 = jnp.dot(feat * inv, Bt_ref[...], preferred_element_type=jnp.float32)
    o_ref[...] = jnp.where(cnt > 0.0, mm + b_ref[...] + mm2, 0.0)


def _tc2_body(h_ref, s2_ref, s1_ref, x_ref, At_ref, Bt_ref, b_ref, o_ref):
    cnt = s1_ref[0][:, D:D + 1] + s1_ref[1][:, D:D + 1]
    s2 = s2_ref[0] + s2_ref[1]
    inv = jnp.where(cnt > 0.0, 1.0 / jnp.maximum(cnt, 1.0), 0.0)
    mm = jnp.dot(h_ref[...], At_ref[...], preferred_element_type=jnp.float32)
    mm2 = jnp.dot(s2 * inv, Bt_ref[...], preferred_element_type=jnp.float32)
    o_ref[...] = jnp.where(cnt > 0.0, mm + b_ref[...] + mm2, 0.0) + x_ref[...]


_W_SPEC = pl.BlockSpec((D, D), lambda i: (0, 0))
_B_SPEC = pl.BlockSpec((1, D), lambda i: (0, 0))
_ROW_SPEC = pl.BlockSpec((BM, D), lambda i: (i, 0))


def _tc1(x, s1_aug, At, Bt, b2d):
    return pl.pallas_call(
        _tc1_body,
        grid=(N_NODES // BM,),
        in_specs=[
            _ROW_SPEC,
            pl.BlockSpec((NC, BM, D_AUG), lambda i: (0, i, 0)),
            _W_SPEC, _W_SPEC, _B_SPEC,
        ],
        out_specs=_ROW_SPEC,
        out_shape=jax.ShapeDtypeStruct((N_NODES, D), jnp.float32),
    )(x, s1_aug, At, Bt, b2d)


def _tc2(h, s2, s1_aug, x, At, Bt, b2d):
    return pl.pallas_call(
        _tc2_body,
        grid=(N_NODES // BM,),
        in_specs=[
            _ROW_SPEC,
            pl.BlockSpec((NC, BM, D), lambda i: (0, i, 0)),
            pl.BlockSpec((NC, BM, D_AUG), lambda i: (0, i, 0)),
            _ROW_SPEC,
            _W_SPEC, _W_SPEC, _B_SPEC,
        ],
        out_specs=_ROW_SPEC,
        out_shape=jax.ShapeDtypeStruct((N_NODES, D), jnp.float32),
    )(h, s2, s1_aug, x, At, Bt, b2d)


def kernel(nodes_feat, edges_index, batch, W_in, b_in, W_out, b_out):
    x = nodes_feat
    src2d = edges_index[0].reshape(N_EDGES // CHUNK, CHUNK)
    dst2d = edges_index[1].reshape(N_EDGES // CHUNK, CHUNK)
    x_aug = jnp.concatenate(
        [x, jnp.ones((N_NODES, 1), x.dtype),
         jnp.zeros((N_NODES, D_AUG - D - 1), x.dtype)], axis=1)
    A1t = (W_in[:, :D] - W_in[:, D:]).T
    B1t = W_in[:, D:].T
    A2t = (W_out[:, :D] - W_out[:, D:]).T
    B2t = W_out[:, D:].T

    s1_aug = _sc_segsum_aug(x_aug, src2d, dst2d)
    h = _tc1(x, s1_aug, A1t, B1t, b_in.reshape(1, D))
    s2 = _sc_segsum_plain(h, src2d, dst2d)
    return _tc2(h, s2, s1_aug, x, A2t, B2t, b_out.reshape(1, D))


# speedup vs baseline: 12.1071x; 12.1071x over previous
import functools

import jax
import jax.numpy as jnp
from jax import lax
from jax.experimental import pallas as pl
from jax.experimental.pallas import tpu as pltpu
from jax.experimental.pallas import tpu_sc as plsc

N_NODES = 10000
N_PAD = 10240
N_EDGES = 320000
D = 128
CW = 16

NC = 2
NS = 16
NW = NC * NS
EDGES_PER_TILE = N_EDGES // NW
CHUNK = 125
CHUNKS_PER_TILE = EDGES_PER_TILE // CHUNK
ROWS_PER_TILE = N_PAD // NS
BM = 400


def _make_sc_segsum(with_cnt):
    mesh = plsc.VectorSubcoreMesh(core_axis_name="c", subcore_axis_name="s")

    out_types = [jax.ShapeDtypeStruct((NC, N_PAD, D), jnp.float32)]
    scratch = [
        pltpu.VMEM((CHUNKS_PER_TILE, CHUNK), jnp.int32),
        pltpu.VMEM((CHUNKS_PER_TILE, CHUNK), jnp.int32),
        pltpu.VMEM((CHUNK, D), jnp.float32),
        pltpu.VMEM_SHARED((N_PAD, D), jnp.float32),
        pltpu.SemaphoreType.DMA,
    ]
    if with_cnt:
        out_types.append(jax.ShapeDtypeStruct((NC, N_PAD, CW), jnp.float32))
        scratch += [
            pltpu.VMEM((CHUNK, CW), jnp.float32),
            pltpu.VMEM_SHARED((N_PAD, CW), jnp.float32),
        ]

    @functools.partial(
        pl.kernel,
        mesh=mesh,
        compiler_params=pltpu.CompilerParams(use_tc_tiling_on_sc=False),
        out_type=out_types,
        scratch_types=scratch,
    )
    def sc_segsum(table_hbm, src_hbm, dst_hbm, zfeat_hbm, *refs):
        if with_cnt:
            (zcnt_hbm, ones_hbm, out_hbm, cnt_hbm,
             src_v, dst_v, rows_v, acc, sem, ones_v, cacc) = refs
        else:
            out_hbm, src_v, dst_v, rows_v, acc, sem = refs
        c = lax.axis_index("c")
        s = lax.axis_index("s")
        wid = s * NC + c
        row0 = s * ROWS_PER_TILE

        pltpu.sync_copy(src_hbm.at[pl.ds(wid * CHUNKS_PER_TILE, CHUNKS_PER_TILE)],
                        src_v)
        pltpu.sync_copy(dst_hbm.at[pl.ds(wid * CHUNKS_PER_TILE, CHUNKS_PER_TILE)],
                        dst_v)
        pltpu.sync_copy(zfeat_hbm.at[pl.ds(row0, ROWS_PER_TILE)],
                        acc.at[pl.ds(row0, ROWS_PER_TILE)])
        if with_cnt:
            pltpu.sync_copy(zcnt_hbm.at[pl.ds(row0, ROWS_PER_TILE)],
                            cacc.at[pl.ds(row0, ROWS_PER_TILE)])
            pltpu.sync_copy(ones_hbm, ones_v)
        plsc.subcore_barrier()

        def _chunk(j, carry):
            pltpu.async_copy(table_hbm.at[src_v.at[j]], rows_v, sem).wait()
            pltpu.sync_copy(rows_v, acc.at[dst_v.at[j]], add=True)
            if with_cnt:
                pltpu.sync_copy(ones_v, cacc.at[dst_v.at[j]], add=True)
            return carry
        lax.fori_loop(0, CHUNKS_PER_TILE, _chunk, 0)
        plsc.subcore_barrier()

        pltpu.sync_copy(acc.at[pl.ds(row0, ROWS_PER_TILE)],
                        out_hbm.at[c, pl.ds(row0, ROWS_PER_TILE)])
        if with_cnt:
            pltpu.sync_copy(cacc.at[pl.ds(row0, ROWS_PER_TILE)],
                            cnt_hbm.at[c, pl.ds(row0, ROWS_PER_TILE)])

    return sc_segsum


_sc_segsum_cnt = _make_sc_segsum(True)
_sc_segsum_plain = _make_sc_segsum(False)


def _tc1_body(x_ref, s_ref, c_ref, At_ref, Bt_ref, b_ref, o_ref):
    cnt = c_ref[0][:, 0:1] + c_ref[1][:, 0:1]
    sblk = s_ref[0] + s_ref[1]
    inv = jnp.where(cnt > 0.0, 1.0 / jnp.maximum(cnt, 1.0), 0.0)
    mm = jnp.dot(x_ref[...], At_ref[...], preferred_element_type=jnp.float32)
    mm2 = jnp.dot(sblk * inv, Bt_ref[...], preferred_element_type=jnp.float32)
    o_ref[...] = jnp.where(cnt > 0.0, mm + b_ref[...] + mm2, 0.0)


def _tc2_body(h_ref, s_ref, c_ref, x_ref, At_ref, Bt_ref, b_ref, o_ref):
    cnt = c_ref[0][:, 0:1] + c_ref[1][:, 0:1]
    sblk = s_ref[0] + s_ref[1]
    inv = jnp.where(cnt > 0.0, 1.0 / jnp.maximum(cnt, 1.0), 0.0)
    mm = jnp.dot(h_ref[...], At_ref[...], preferred_element_type=jnp.float32)
    mm2 = jnp.dot(sblk * inv, Bt_ref[...], preferred_element_type=jnp.float32)
    o_ref[...] = jnp.where(cnt > 0.0, mm + b_ref[...] + mm2, 0.0) + x_ref[...]


_W_SPEC = pl.BlockSpec((D, D), lambda i: (0, 0))
_B_SPEC = pl.BlockSpec((1, D), lambda i: (0, 0))
_ROW_SPEC = pl.BlockSpec((BM, D), lambda i: (i, 0))
_S_SPEC = pl.BlockSpec((NC, BM, D), lambda i: (0, i, 0))
_C_SPEC = pl.BlockSpec((NC, BM, CW), lambda i: (0, i, 0))


def _tc1(x, s1, cnt_p, At, Bt, b2d):
    return pl.pallas_call(
        _tc1_body,
        grid=(N_NODES // BM,),
        in_specs=[_ROW_SPEC, _S_SPEC, _C_SPEC, _W_SPEC, _W_SPEC, _B_SPEC],
        out_specs=_ROW_SPEC,
        out_shape=jax.ShapeDtypeStruct((N_NODES, D), jnp.float32),
    )(x, s1, cnt_p, At, Bt, b2d)


def _tc2(h, s2, cnt_p, x, At, Bt, b2d):
    return pl.pallas_call(
        _tc2_body,
        grid=(N_NODES // BM,),
        in_specs=[_ROW_SPEC, _S_SPEC, _C_SPEC, _ROW_SPEC,
                  _W_SPEC, _W_SPEC, _B_SPEC],
        out_specs=_ROW_SPEC,
        out_shape=jax.ShapeDtypeStruct((N_NODES, D), jnp.float32),
    )(h, s2, cnt_p, x, At, Bt, b2d)


def kernel(nodes_feat, edges_index, batch, W_in, b_in, W_out, b_out):
    x = nodes_feat
    src2d = edges_index[0].reshape(N_EDGES // CHUNK, CHUNK)
    dst2d = edges_index[1].reshape(N_EDGES // CHUNK, CHUNK)
    zfeat = jnp.zeros((N_PAD, D), jnp.float32)
    zcnt = jnp.zeros((N_PAD, CW), jnp.float32)
    ones2d = jnp.ones((CHUNK, CW), jnp.float32)
    A1t = (W_in[:, :D] - W_in[:, D:]).T
    B1t = W_in[:, D:].T
    A2t = (W_out[:, :D] - W_out[:, D:]).T
    B2t = W_out[:, D:].T

    s1, cnt_p = _sc_segsum_cnt(x, src2d, dst2d, zfeat, zcnt, ones2d)
    h = _tc1(x, s1, cnt_p, A1t, B1t, b_in.reshape(1, D))
    (s2,) = _sc_segsum_plain(h, src2d, dst2d, zfeat)
    return _tc2(h, s2, cnt_p, x, A2t, B2t, b_out.reshape(1, D))


# speedup vs baseline: 16.9489x; 1.3999x over previous
import functools

import jax
import jax.numpy as jnp
from jax import lax
from jax.experimental import pallas as pl
from jax.experimental.pallas import tpu as pltpu
from jax.experimental.pallas import tpu_sc as plsc

N_NODES = 10000
N_PAD = 10240
N_EDGES = 320000
D = 128
CW = 16

NC = 2
NS = 16
NW = NC * NS
EDGES_PER_TILE = N_EDGES // NW
CHUNK = 100
CHUNKS_PER_TILE = EDGES_PER_TILE // CHUNK
ROWS_PER_TILE = N_PAD // NS
NBUF = 2
BM = 400

_MESH = plsc.VectorSubcoreMesh(core_axis_name="c", subcore_axis_name="s")
_SC_PARAMS = pltpu.CompilerParams(use_tc_tiling_on_sc=False)


def _tile_ids():
    c = lax.axis_index("c")
    s = lax.axis_index("s")
    return c, s, s * NC + c


def _stage_idx(idx_hbm, idx_v, wid):
    pltpu.sync_copy(idx_hbm.at[pl.ds(wid * CHUNKS_PER_TILE, CHUNKS_PER_TILE)],
                    idx_v)


@functools.partial(
    pl.kernel,
    mesh=_MESH,
    compiler_params=_SC_PARAMS,
    out_type=jax.ShapeDtypeStruct((NC, N_PAD, D), jnp.float32),
    scratch_types=[
        pltpu.VMEM((CHUNKS_PER_TILE, CHUNK), jnp.int32),
        pltpu.VMEM((CHUNKS_PER_TILE, CHUNK), jnp.int32),
        [pltpu.VMEM((CHUNK, D), jnp.float32)] * NBUF,
        pltpu.VMEM_SHARED((N_PAD, D), jnp.float32),
        [pltpu.SemaphoreType.DMA] * NBUF,
    ],
)
def _sc_segsum(table_hbm, src_hbm, dst_hbm, zfeat_hbm, out_hbm,
               src_v, dst_v, rows, acc, sems):
    c, s, wid = _tile_ids()
    row0 = s * ROWS_PER_TILE
    _stage_idx(src_hbm, src_v, wid)
    _stage_idx(dst_hbm, dst_v, wid)
    pltpu.sync_copy(zfeat_hbm.at[pl.ds(row0, ROWS_PER_TILE)],
                    acc.at[pl.ds(row0, ROWS_PER_TILE)])
    plsc.subcore_barrier()

    for b in range(NBUF - 1):
        pltpu.async_copy(table_hbm.at[src_v.at[b]], rows[b], sems[b])

    def _group(i, carry):
        for b in range(NBUF):
            j = i * NBUF + b
            jn = j + NBUF - 1
            bn = (NBUF - 1 + b) % NBUF

            @pl.when(jn < CHUNKS_PER_TILE)
            def _():
                pltpu.async_copy(table_hbm.at[src_v.at[jn]], rows[bn],
                                 sems[bn])
            pltpu.make_async_copy(table_hbm.at[src_v.at[j]], rows[b],
                                  sems[b]).wait()
            pltpu.sync_copy(rows[b], acc.at[dst_v.at[j]], add=True)
        return carry
    lax.fori_loop(0, CHUNKS_PER_TILE // NBUF, _group, 0)
    plsc.subcore_barrier()

    pltpu.sync_copy(acc.at[pl.ds(row0, ROWS_PER_TILE)],
                    out_hbm.at[c, pl.ds(row0, ROWS_PER_TILE)])


@functools.partial(
    pl.kernel,
    mesh=_MESH,
    compiler_params=_SC_PARAMS,
    out_type=jax.ShapeDtypeStruct((NC, N_PAD, CW), jnp.float32),
    scratch_types=[
        pltpu.VMEM((CHUNKS_PER_TILE, CHUNK), jnp.int32),
        pltpu.VMEM((CHUNK, CW), jnp.float32),
        pltpu.VMEM_SHARED((N_PAD, CW), jnp.float32),
    ],
)
def _sc_count(dst_hbm, zcnt_hbm, ones_hbm, cnt_hbm, dst_v, ones_v, cacc):
    c, s, wid = _tile_ids()
    row0 = s * ROWS_PER_TILE
    _stage_idx(dst_hbm, dst_v, wid)
    pltpu.sync_copy(ones_hbm, ones_v)
    pltpu.sync_copy(zcnt_hbm.at[pl.ds(row0, ROWS_PER_TILE)],
                    cacc.at[pl.ds(row0, ROWS_PER_TILE)])
    plsc.subcore_barrier()

    def _chunk(j, carry):
        pltpu.sync_copy(ones_v, cacc.at[dst_v.at[j]], add=True)
        return carry
    lax.fori_loop(0, CHUNKS_PER_TILE, _chunk, 0)
    plsc.subcore_barrier()

    pltpu.sync_copy(cacc.at[pl.ds(row0, ROWS_PER_TILE)],
                    cnt_hbm.at[c, pl.ds(row0, ROWS_PER_TILE)])


def _tc1_body(x_ref, s_ref, c_ref, At_ref, Bt_ref, b_ref, o_ref):
    cnt = c_ref[0][:, 0:1] + c_ref[1][:, 0:1]
    sblk = s_ref[0] + s_ref[1]
    inv = jnp.where(cnt > 0.0, 1.0 / jnp.maximum(cnt, 1.0), 0.0)
    mm = jnp.dot(x_ref[...], At_ref[...], preferred_element_type=jnp.float32)
    mm2 = jnp.dot(sblk * inv, Bt_ref[...], preferred_element_type=jnp.float32)
    o_ref[...] = jnp.where(cnt > 0.0, mm + b_ref[...] + mm2, 0.0)


def _tc2_body(h_ref, s_ref, c_ref, x_ref, At_ref, Bt_ref, b_ref, o_ref):
    cnt = c_ref[0][:, 0:1] + c_ref[1][:, 0:1]
    sblk = s_ref[0] + s_ref[1]
    inv = jnp.where(cnt > 0.0, 1.0 / jnp.maximum(cnt, 1.0), 0.0)
    mm = jnp.dot(h_ref[...], At_ref[...], preferred_element_type=jnp.float32)
    mm2 = jnp.dot(sblk * inv, Bt_ref[...], preferred_element_type=jnp.float32)
    o_ref[...] = jnp.where(cnt > 0.0, mm + b_ref[...] + mm2, 0.0) + x_ref[...]


_W_SPEC = pl.BlockSpec((D, D), lambda i: (0, 0))
_B_SPEC = pl.BlockSpec((1, D), lambda i: (0, 0))
_ROW_SPEC = pl.BlockSpec((BM, D), lambda i: (i, 0))
_S_SPEC = pl.BlockSpec((NC, BM, D), lambda i: (0, i, 0))
_C_SPEC = pl.BlockSpec((NC, BM, CW), lambda i: (0, i, 0))


def _tc1(x, s1, cnt_p, At, Bt, b2d):
    return pl.pallas_call(
        _tc1_body,
        grid=(N_NODES // BM,),
        in_specs=[_ROW_SPEC, _S_SPEC, _C_SPEC, _W_SPEC, _W_SPEC, _B_SPEC],
        out_specs=_ROW_SPEC,
        out_shape=jax.ShapeDtypeStruct((N_NODES, D), jnp.float32),
    )(x, s1, cnt_p, At, Bt, b2d)


def _tc2(h, s2, cnt_p, x, At, Bt, b2d):
    return pl.pallas_call(
        _tc2_body,
        grid=(N_NODES // BM,),
        in_specs=[_ROW_SPEC, _S_SPEC, _C_SPEC, _ROW_SPEC,
                  _W_SPEC, _W_SPEC, _B_SPEC],
        out_specs=_ROW_SPEC,
        out_shape=jax.ShapeDtypeStruct((N_NODES, D), jnp.float32),
    )(h, s2, cnt_p, x, At, Bt, b2d)


def kernel(nodes_feat, edges_index, batch, W_in, b_in, W_out, b_out):
    x = nodes_feat
    src2d = edges_index[0].reshape(N_EDGES // CHUNK, CHUNK)
    dst2d = edges_index[1].reshape(N_EDGES // CHUNK, CHUNK)
    zfeat = jnp.zeros((N_PAD, D), jnp.float32)
    zcnt = jnp.zeros((N_PAD, CW), jnp.float32)
    ones2d = jnp.ones((CHUNK, CW), jnp.float32)
    A1t = (W_in[:, :D] - W_in[:, D:]).T
    B1t = W_in[:, D:].T
    A2t = (W_out[:, :D] - W_out[:, D:]).T
    B2t = W_out[:, D:].T

    cnt_p = _sc_count(dst2d, zcnt, ones2d)
    s1 = _sc_segsum(x, src2d, dst2d, zfeat)
    h = _tc1(x, s1, cnt_p, A1t, B1t, b_in.reshape(1, D))
    s2 = _sc_segsum(h, src2d, dst2d, zfeat)
    return _tc2(h, s2, cnt_p, x, A2t, B2t, b_out.reshape(1, D))


# speedup vs baseline: 18.0908x; 1.0674x over previous
import functools

import jax
import jax.numpy as jnp
from jax import lax
from jax.experimental import pallas as pl
from jax.experimental.pallas import tpu as pltpu
from jax.experimental.pallas import tpu_sc as plsc

N_NODES = 10000
N_PAD = 10240
N_EDGES = 320000
D = 128
CW = 16

NC = 2
NS = 16
NW = NC * NS
EDGES_PER_TILE = N_EDGES // NW
CHUNK = 50
CHUNKS_PER_TILE = EDGES_PER_TILE // CHUNK
ROWS_PER_TILE = N_PAD // NS
NBUF = 4
BM = 400

_MESH = plsc.VectorSubcoreMesh(core_axis_name="c", subcore_axis_name="s")
_SC_PARAMS = pltpu.CompilerParams(use_tc_tiling_on_sc=False)


def _tile_ids():
    c = lax.axis_index("c")
    s = lax.axis_index("s")
    return c, s, s * NC + c


def _stage_idx(idx_hbm, idx_v, wid):
    pltpu.sync_copy(idx_hbm.at[pl.ds(wid * CHUNKS_PER_TILE, CHUNKS_PER_TILE)],
                    idx_v)


@functools.partial(
    pl.kernel,
    mesh=_MESH,
    compiler_params=_SC_PARAMS,
    out_type=jax.ShapeDtypeStruct((NC, N_PAD, D), jnp.float32),
    scratch_types=[
        pltpu.VMEM((CHUNKS_PER_TILE, CHUNK), jnp.int32),
        pltpu.VMEM((CHUNKS_PER_TILE, CHUNK), jnp.int32),
        [pltpu.VMEM((CHUNK, D), jnp.float32)] * NBUF,
        pltpu.VMEM_SHARED((N_PAD, D), jnp.float32),
        [pltpu.SemaphoreType.DMA] * NBUF,
    ],
)
def _sc_segsum(table_hbm, src_hbm, dst_hbm, zfeat_hbm, out_hbm,
               src_v, dst_v, rows, acc, sems):
    c, s, wid = _tile_ids()
    row0 = s * ROWS_PER_TILE
    _stage_idx(src_hbm, src_v, wid)
    _stage_idx(dst_hbm, dst_v, wid)
    pltpu.sync_copy(zfeat_hbm.at[pl.ds(row0, ROWS_PER_TILE)],
                    acc.at[pl.ds(row0, ROWS_PER_TILE)])
    plsc.subcore_barrier()

    for b in range(NBUF - 1):
        pltpu.async_copy(table_hbm.at[src_v.at[b]], rows[b], sems[b])

    def _group(i, carry):
        for b in range(NBUF):
            j = i * NBUF + b
            jn = j + NBUF - 1
            bn = (NBUF - 1 + b) % NBUF

            @pl.when(jn < CHUNKS_PER_TILE)
            def _():
                pltpu.async_copy(table_hbm.at[src_v.at[jn]], rows[bn],
                                 sems[bn])
            pltpu.make_async_copy(table_hbm.at[src_v.at[j]], rows[b],
                                  sems[b]).wait()
            pltpu.sync_copy(rows[b], acc.at[dst_v.at[j]], add=True)
        return carry
    lax.fori_loop(0, CHUNKS_PER_TILE // NBUF, _group, 0)
    plsc.subcore_barrier()

    pltpu.sync_copy(acc.at[pl.ds(row0, ROWS_PER_TILE)],
                    out_hbm.at[c, pl.ds(row0, ROWS_PER_TILE)])


@functools.partial(
    pl.kernel,
    mesh=_MESH,
    compiler_params=_SC_PARAMS,
    out_type=jax.ShapeDtypeStruct((NC, N_PAD, CW), jnp.float32),
    scratch_types=[
        pltpu.VMEM((CHUNKS_PER_TILE, CHUNK), jnp.int32),
        pltpu.VMEM((CHUNK, CW), jnp.float32),
        pltpu.VMEM_SHARED((N_PAD, CW), jnp.float32),
    ],
)
def _sc_count(dst_hbm, zcnt_hbm, ones_hbm, cnt_hbm, dst_v, ones_v, cacc):
    c, s, wid = _tile_ids()
    row0 = s * ROWS_PER_TILE
    _stage_idx(dst_hbm, dst_v, wid)
    pltpu.sync_copy(ones_hbm, ones_v)
    pltpu.sync_copy(zcnt_hbm.at[pl.ds(row0, ROWS_PER_TILE)],
                    cacc.at[pl.ds(row0, ROWS_PER_TILE)])
    plsc.subcore_barrier()

    def _chunk(j, carry):
        pltpu.sync_copy(ones_v, cacc.at[dst_v.at[j]], add=True)
        return carry
    lax.fori_loop(0, CHUNKS_PER_TILE, _chunk, 0)
    plsc.subcore_barrier()

    pltpu.sync_copy(cacc.at[pl.ds(row0, ROWS_PER_TILE)],
                    cnt_hbm.at[c, pl.ds(row0, ROWS_PER_TILE)])


def _tc1_body(x_ref, s_ref, c_ref, At_ref, Bt_ref, b_ref, o_ref):
    cnt = c_ref[0][:, 0:1] + c_ref[1][:, 0:1]
    sblk = s_ref[0] + s_ref[1]
    inv = jnp.where(cnt > 0.0, 1.0 / jnp.maximum(cnt, 1.0), 0.0)
    mm = jnp.dot(x_ref[...], At_ref[...], preferred_element_type=jnp.float32)
    mm2 = jnp.dot(sblk * inv, Bt_ref[...], preferred_element_type=jnp.float32)
    o_ref[...] = jnp.where(cnt > 0.0, mm + b_ref[...] + mm2, 0.0)


def _tc2_body(h_ref, s_ref, c_ref, x_ref, At_ref, Bt_ref, b_ref, o_ref):
    cnt = c_ref[0][:, 0:1] + c_ref[1][:, 0:1]
    sblk = s_ref[0] + s_ref[1]
    inv = jnp.where(cnt > 0.0, 1.0 / jnp.maximum(cnt, 1.0), 0.0)
    mm = jnp.dot(h_ref[...], At_ref[...], preferred_element_type=jnp.float32)
    mm2 = jnp.dot(sblk * inv, Bt_ref[...], preferred_element_type=jnp.float32)
    o_ref[...] = jnp.where(cnt > 0.0, mm + b_ref[...] + mm2, 0.0) + x_ref[...]


_W_SPEC = pl.BlockSpec((D, D), lambda i: (0, 0))
_B_SPEC = pl.BlockSpec((1, D), lambda i: (0, 0))
_ROW_SPEC = pl.BlockSpec((BM, D), lambda i: (i, 0))
_S_SPEC = pl.BlockSpec((NC, BM, D), lambda i: (0, i, 0))
_C_SPEC = pl.BlockSpec((NC, BM, CW), lambda i: (0, i, 0))


def _tc1(x, s1, cnt_p, At, Bt, b2d):
    return pl.pallas_call(
        _tc1_body,
        grid=(N_NODES // BM,),
        in_specs=[_ROW_SPEC, _S_SPEC, _C_SPEC, _W_SPEC, _W_SPEC, _B_SPEC],
        out_specs=_ROW_SPEC,
        out_shape=jax.ShapeDtypeStruct((N_NODES, D), jnp.float32),
    )(x, s1, cnt_p, At, Bt, b2d)


def _tc2(h, s2, cnt_p, x, At, Bt, b2d):
    return pl.pallas_call(
        _tc2_body,
        grid=(N_NODES // BM,),
        in_specs=[_ROW_SPEC, _S_SPEC, _C_SPEC, _ROW_SPEC,
                  _W_SPEC, _W_SPEC, _B_SPEC],
        out_specs=_ROW_SPEC,
        out_shape=jax.ShapeDtypeStruct((N_NODES, D), jnp.float32),
    )(h, s2, cnt_p, x, At, Bt, b2d)


def kernel(nodes_feat, edges_index, batch, W_in, b_in, W_out, b_out):
    x = nodes_feat
    src2d = edges_index[0].reshape(N_EDGES // CHUNK, CHUNK)
    dst2d = edges_index[1].reshape(N_EDGES // CHUNK, CHUNK)
    zfeat = jnp.zeros((N_PAD, D), jnp.float32)
    zcnt = jnp.zeros((N_PAD, CW), jnp.float32)
    ones2d = jnp.ones((CHUNK, CW), jnp.float32)
    A1t = (W_in[:, :D] - W_in[:, D:]).T
    B1t = W_in[:, D:].T
    A2t = (W_out[:, :D] - W_out[:, D:]).T
    B2t = W_out[:, D:].T

    cnt_p = _sc_count(dst2d, zcnt, ones2d)
    s1 = _sc_segsum(x, src2d, dst2d, zfeat)
    h = _tc1(x, s1, cnt_p, A1t, B1t, b_in.reshape(1, D))
    s2 = _sc_segsum(h, src2d, dst2d, zfeat)
    return _tc2(h, s2, cnt_p, x, A2t, B2t, b_out.reshape(1, D))
